# baseline (speedup 1.0000x reference)
import functools
import math

import jax
import jax.numpy as jnp
from jax import lax
from jax.experimental import pallas as pl
from jax.experimental.pallas import tpu as pltpu
from jax.experimental.pallas import tpu_sc as plsc

_NUM_CORES = 2
_NUM_SUBCORES = 16
_LANES = 16
_NUM_WORKERS = _NUM_CORES * _NUM_SUBCORES


@functools.partial(jax.jit, static_argnames=("batch", "seq", "d"))
def _embed_sc(x_flat, table, pe, *, batch, seq, d):
    s_per_w = seq // _NUM_WORKERS
    chunk = 32
    n_chunks = s_per_w // chunk
    scale = float(math.sqrt(d))
    groups = d // _LANES

    mesh = plsc.VectorSubcoreMesh(
        core_axis_name="c", subcore_axis_name="s"
    )

    @functools.partial(
        pl.kernel,
        out_type=jax.ShapeDtypeStruct((batch * seq, d), jnp.float32),
        mesh=mesh,
        scratch_types=[
            pltpu.VMEM((batch * s_per_w,), jnp.int32),
            pltpu.VMEM((chunk, d), jnp.float32),
            pltpu.VMEM((chunk, d), jnp.float32),
            pltpu.SemaphoreType.DMA,
        ],
    )
    def k(x_hbm, table_hbm, pe_hbm, out_hbm, idx_v, rows_v, pe_v, sem):
        wid = lax.axis_index("s") * _NUM_CORES + lax.axis_index("c")
        s0 = wid * s_per_w
        for b in range(batch):
            pltpu.sync_copy(
                x_hbm.at[pl.ds(b * seq + s0, s_per_w)],
                idx_v.at[pl.ds(b * s_per_w, s_per_w)],
            )
        for c in range(n_chunks):
            sc0 = s0 + c * chunk
            pltpu.sync_copy(pe_hbm.at[pl.ds(sc0, chunk)], pe_v)
            for b in range(batch):
                pltpu.async_copy(
                    table_hbm.at[idx_v.at[pl.ds(b * s_per_w + c * chunk, chunk)]],
                    rows_v,
                    sem,
                ).wait()

                def row_body(r, _):
                    for j in range(groups):
                        v = rows_v[r, pl.ds(j * _LANES, _LANES)]
                        p = pe_v[r, pl.ds(j * _LANES, _LANES)]
                        rows_v[r, pl.ds(j * _LANES, _LANES)] = v * scale + p
                    return 0

                lax.fori_loop(0, chunk, row_body, 0)
                pltpu.sync_copy(
                    rows_v, out_hbm.at[pl.ds(b * seq + sc0, chunk)]
                )

    return k(x_flat, table, pe)


def kernel(x, table, pe):
    batch, seq = x.shape
    d = table.shape[1]
    x_flat = x.reshape(-1).astype(jnp.int32)
    out = _embed_sc(x_flat, table, pe, batch=batch, seq=seq, d=d)
    return out.reshape(batch, seq, d)


# speedup vs baseline: 1.2304x; 1.2304x over previous
import functools
import math

import jax
import jax.numpy as jnp
from jax import lax
from jax.experimental import pallas as pl
from jax.experimental.pallas import tpu as pltpu
from jax.experimental.pallas import tpu_sc as plsc

_NUM_CORES = 2
_NUM_SUBCORES = 16
_LANES = 16
_NUM_WORKERS = _NUM_CORES * _NUM_SUBCORES


@functools.partial(jax.jit, static_argnames=("batch", "seq", "d"))
def _embed_sc(x_flat, table, pe, *, batch, seq, d):
    s_per_w = seq // _NUM_WORKERS
    chunk = 32
    n_chunks = s_per_w // chunk
    scale = float(math.sqrt(d))
    groups = d // _LANES

    mesh = plsc.VectorSubcoreMesh(
        core_axis_name="c", subcore_axis_name="s"
    )

    n_steps = n_chunks * batch

    @functools.partial(
        pl.kernel,
        out_type=jax.ShapeDtypeStruct((batch * seq, d), jnp.float32),
        mesh=mesh,
        scratch_types=[
            pltpu.VMEM((batch * s_per_w,), jnp.int32),
            pltpu.VMEM((chunk, d), jnp.float32),
            pltpu.VMEM((chunk, d), jnp.float32),
            pltpu.VMEM((chunk, d), jnp.float32),
            pltpu.SemaphoreType.DMA,
            pltpu.SemaphoreType.DMA,
            pltpu.SemaphoreType.DMA,
            pltpu.SemaphoreType.DMA,
        ],
    )
    def k(x_hbm, table_hbm, pe_hbm, out_hbm, idx_v, rows0, rows1, pe_v,
          g0, g1, s0sem, s1sem):
        rows = (rows0, rows1)
        gsem = (g0, g1)
        ssem = (s0sem, s1sem)
        wid = lax.axis_index("s") * _NUM_CORES + lax.axis_index("c")
        s0 = wid * s_per_w
        for b in range(batch):
            pltpu.sync_copy(
                x_hbm.at[pl.ds(b * seq + s0, s_per_w)],
                idx_v.at[pl.ds(b * s_per_w, s_per_w)],
            )

        def gather(t, buf):
            c, b = divmod(t, batch)
            return pltpu.async_copy(
                table_hbm.at[idx_v.at[pl.ds(b * s_per_w + c * chunk, chunk)]],
                rows[buf],
                gsem[buf],
            )

        def store(t, buf):
            c, b = divmod(t, batch)
            return pltpu.async_copy(
                rows[buf],
                out_hbm.at[pl.ds(b * seq + s0 + c * chunk, chunk)],
                ssem[buf],
            )

        gathers = [gather(0, 0), None]
        stores = [None, None]
        for t in range(n_steps):
            buf = t % 2
            nxt = (t + 1) % 2
            if t + 1 < n_steps:
                if stores[nxt] is not None:
                    stores[nxt].wait()
                gathers[nxt] = gather(t + 1, nxt)
            if t % batch == 0:
                pltpu.sync_copy(
                    pe_hbm.at[pl.ds(s0 + (t // batch) * chunk, chunk)], pe_v
                )
            gathers[buf].wait()

            rv = rows[buf]

            def row_body(r, _):
                for j in range(groups):
                    v = rv[r, pl.ds(j * _LANES, _LANES)]
                    p = pe_v[r, pl.ds(j * _LANES, _LANES)]
                    rv[r, pl.ds(j * _LANES, _LANES)] = v * scale + p
                return 0

            lax.fori_loop(0, chunk, row_body, 0)
            stores[buf] = store(t, buf)
        stores[0].wait()
        if stores[1] is not None:
            stores[1].wait()

    return k(x_flat, table, pe)


def kernel(x, table, pe):
    batch, seq = x.shape
    d = table.shape[1]
    x_flat = x.reshape(-1).astype(jnp.int32)
    out = _embed_sc(x_flat, table, pe, batch=batch, seq=seq, d=d)
    return out.reshape(batch, seq, d)


# speedup vs baseline: 1.5617x; 1.2692x over previous
import functools
import math

import jax
import jax.numpy as jnp
from jax import lax
from jax.experimental import pallas as pl
from jax.experimental.pallas import tpu as pltpu
from jax.experimental.pallas import tpu_sc as plsc

_NUM_CORES = 2
_NUM_SUBCORES = 16
_LANES = 16
_NUM_WORKERS = _NUM_CORES * _NUM_SUBCORES

_NBUF = 4
_LOOKAHEAD = 2


@functools.partial(jax.jit, static_argnames=("batch", "seq", "d"))
def _embed_sc(x_flat, table, pe, *, batch, seq, d):
    s_per_w = seq // _NUM_WORKERS
    chunk = 16
    n_chunks = s_per_w // chunk
    scale = float(math.sqrt(d))
    groups = d // _LANES
    n_steps = n_chunks * batch

    mesh = plsc.VectorSubcoreMesh(core_axis_name="c", subcore_axis_name="s")

    @functools.partial(
        pl.kernel,
        out_type=jax.ShapeDtypeStruct((batch * seq, d), jnp.float32),
        mesh=mesh,
        scratch_types=[
            pltpu.VMEM((batch * s_per_w,), jnp.int32),
            [pltpu.VMEM((chunk, d), jnp.float32)] * _NBUF,
            [pltpu.VMEM((chunk, d), jnp.float32)] * 2,
            [pltpu.SemaphoreType.DMA] * _NBUF,
            [pltpu.SemaphoreType.DMA] * _NBUF,
            [pltpu.SemaphoreType.DMA] * 2,
        ],
    )
    def k(x_hbm, table_hbm, pe_hbm, out_hbm, idx_v, rows, pes, gsem, ssem,
          psem):
        wid = lax.axis_index("s") * _NUM_CORES + lax.axis_index("c")
        s0 = wid * s_per_w
        for b in range(batch):
            pltpu.sync_copy(
                x_hbm.at[pl.ds(b * seq + s0, s_per_w)],
                idx_v.at[pl.ds(b * s_per_w, s_per_w)],
            )

        def gather(t):
            c, b = divmod(t, batch)
            return pltpu.async_copy(
                table_hbm.at[idx_v.at[pl.ds(b * s_per_w + c * chunk, chunk)]],
                rows[t % _NBUF],
                gsem[t % _NBUF],
            )

        def store(t):
            c, b = divmod(t, batch)
            return pltpu.async_copy(
                rows[t % _NBUF],
                out_hbm.at[pl.ds(b * seq + s0 + c * chunk, chunk)],
                ssem[t % _NBUF],
            )

        def load_pe(c):
            return pltpu.async_copy(
                pe_hbm.at[pl.ds(s0 + c * chunk, chunk)],
                pes[c % 2],
                psem[c % 2],
            )

        gathers = [None] * _NBUF
        stores = [None] * _NBUF
        pe_loads = [None, None]
        pe_loads[0] = load_pe(0)
        pe_loads[1] = load_pe(1)
        for t in range(_LOOKAHEAD):
            gathers[t % _NBUF] = gather(t)

        for t in range(n_steps):
            buf = t % _NBUF
            ta = t + _LOOKAHEAD
            if ta < n_steps:
                if stores[ta % _NBUF] is not None:
                    stores[ta % _NBUF].wait()
                gathers[ta % _NBUF] = gather(ta)
            c, b = divmod(t, batch)
            if b == 0:
                pe_loads[c % 2].wait()
            pv = pes[c % 2]
            gathers[buf].wait()
            rv = rows[buf]

            def row_body(r, _):
                for j in range(groups):
                    v = rv[r, pl.ds(j * _LANES, _LANES)]
                    p = pv[r, pl.ds(j * _LANES, _LANES)]
                    rv[r, pl.ds(j * _LANES, _LANES)] = v * scale + p
                return 0

            lax.fori_loop(0, chunk, row_body, 0)
            stores[buf] = store(t)
            if b == batch - 1 and c + 2 < n_chunks:
                pe_loads[c % 2] = load_pe(c + 2)

        for s in stores:
            if s is not None:
                s.wait()

    return k(x_flat, table, pe)


def kernel(x, table, pe):
    batch, seq = x.shape
    d = table.shape[1]
    x_flat = x.reshape(-1).astype(jnp.int32)
    out = _embed_sc(x_flat, table, pe, batch=batch, seq=seq, d=d)
    return out.reshape(batch, seq, d)
